# baseline (speedup 1.0000x reference)
import functools

import jax
import jax.numpy as jnp
from jax.experimental import pallas as pl
from jax.experimental.pallas import tpu as pltpu

B = 64
V = 100000
CHUNK = 1280
NCHUNK = (V + CHUNK - 1) // CHUNK


def _logprob_kernel(logits_ref, m_ref, s_ref, out_ref):
    x = logits_ref[...]
    out_ref[...] = x - m_ref[...] - jnp.log(s_ref[...])


def kernel(logits, top_ks, top_ps, min_ps):
    m = jnp.max(logits, axis=-1, keepdims=True)
    s = jnp.sum(jnp.exp(logits - m), axis=-1, keepdims=True)

    logprobs = pl.pallas_call(
        _logprob_kernel,
        grid=(NCHUNK,),
        in_specs=[
            pl.BlockSpec((B, CHUNK), lambda i: (0, i)),
            pl.BlockSpec((B, 1), lambda i: (0, 0)),
            pl.BlockSpec((B, 1), lambda i: (0, 0)),
        ],
        out_specs=pl.BlockSpec((B, CHUNK), lambda i: (0, i)),
        out_shape=jax.ShapeDtypeStruct((B, V), jnp.float32),
    )(logits, m, s)

    probs = jax.nn.softmax(logits, axis=-1)
    sort_idx = jnp.argsort(-probs, axis=-1)
    probs_sort = jnp.take_along_axis(probs, sort_idx, axis=-1)
    probs_sum = jnp.cumsum(probs_sort, axis=-1)
    ar = jnp.arange(V, dtype=jnp.int32)[None, :]
    probs_sort = jnp.where(ar >= top_ks[:, None], 0.0, probs_sort)
    probs_sort = jnp.where(probs_sum - probs_sort > top_ps[:, None], 0.0, probs_sort)
    min_p_thresholds = probs_sort[:, 0] * min_ps
    probs_sort = jnp.where(probs_sort < min_p_thresholds[:, None], 0.0, probs_sort)
    skey = jax.random.fold_in(jax.random.key(0), 12345)
    sampled_index = jax.random.categorical(skey, jnp.log(probs_sort + 1e-30), axis=-1)
    probs_idx = sort_idx.astype(jnp.int32)
    batch_next_token_ids = jnp.take_along_axis(
        probs_idx, sampled_index[:, None].astype(jnp.int32), axis=1
    ).reshape(-1)
    return (batch_next_token_ids, logprobs)


# speedup vs baseline: 35.3381x; 35.3381x over previous
import functools

import numpy as np
import jax
import jax.numpy as jnp
from jax import lax
from jax.experimental import pallas as pl
from jax.experimental.pallas import tpu as pltpu
from jax.experimental.pallas import tpu_sc as plsc

B = 64
V = 100000
K = 1024
CHUNK = 1280
NCHUNK = (V + CHUNK - 1) // CHUNK
SUB = CHUNK // 128


def _threefry2x32_np(k1, k2, x0, x1):
    rot = [[13, 15, 26, 6], [17, 29, 16, 24]]
    ks = [np.uint32(k1), np.uint32(k2),
          np.uint32(k1) ^ np.uint32(k2) ^ np.uint32(0x1BD11BDA)]
    x = [(x0 + ks[0]).astype(np.uint32), (x1 + ks[1]).astype(np.uint32)]

    def rounds(x, rs):
        for r in rs:
            x[0] = (x[0] + x[1]).astype(np.uint32)
            x[1] = (x[1] << np.uint32(r)) | (x[1] >> np.uint32(32 - r))
            x[1] = x[0] ^ x[1]
        return x

    x = rounds(x, rot[0]); x[0] = x[0] + ks[1]; x[1] = x[1] + ks[2] + np.uint32(1)
    x = rounds(x, rot[1]); x[0] = x[0] + ks[2]; x[1] = x[1] + ks[0] + np.uint32(2)
    x = rounds(x, rot[0]); x[0] = x[0] + ks[0]; x[1] = x[1] + ks[1] + np.uint32(3)
    x = rounds(x, rot[1]); x[0] = x[0] + ks[1]; x[1] = x[1] + ks[2] + np.uint32(4)
    x = rounds(x, rot[0]); x[0] = x[0] + ks[2]; x[1] = x[1] + ks[0] + np.uint32(5)
    return x[0].astype(np.uint32), x[1].astype(np.uint32)


def _make_uniform_slab():
    old = np.seterr(over="ignore")
    try:
        kk1, kk2 = _threefry2x32_np(
            np.uint32(0), np.uint32(0),
            np.array([0], dtype=np.uint32), np.array([12345], dtype=np.uint32))
        f = (np.arange(B, dtype=np.uint64)[:, None] * np.uint64(V)
             + np.arange(K, dtype=np.uint64)[None, :])
        c1 = (f >> np.uint64(32)).astype(np.uint32)
        c2 = (f & np.uint64(0xFFFFFFFF)).astype(np.uint32)
        o1, o2 = _threefry2x32_np(kk1[0], kk2[0], c1, c2)
        bits = o1 ^ o2
    finally:
        np.seterr(**old)
    fb = (bits >> np.uint32(9)) | np.uint32(0x3F800000)
    floats = fb.view(np.float32) - np.float32(1.0)
    tiny = np.float32(np.finfo(np.float32).tiny)
    return np.maximum(tiny, (floats * np.float32(np.float32(1.0) - tiny)
                             + tiny).astype(np.float32))


_U_SLAB = _make_uniform_slab()


def _stats_kernel(x_ref, m_ref, s_ref, accm_ref, accs_ref):
    i = pl.program_id(0)

    @pl.when(i == 0)
    def _():
        accm_ref[...] = jnp.full_like(accm_ref, -jnp.inf)
        accs_ref[...] = jnp.zeros_like(accs_ref)

    x = x_ref[...]
    neg = jnp.float32(-jnp.inf)
    for t in range(SUB):
        col = i * CHUNK + t * 128 + lax.broadcasted_iota(jnp.int32, (B, 128), 1)
        xt = jnp.where(col < V, x[:, t * 128:(t + 1) * 128], neg)
        m_old = accm_ref[...]
        m_new = jnp.maximum(m_old, xt)
        accs_ref[...] = (accs_ref[...] * jnp.exp(m_old - m_new)
                         + jnp.exp(xt - m_new))
        accm_ref[...] = m_new

    @pl.when(i == NCHUNK - 1)
    def _():
        m = jnp.max(accm_ref[...], axis=-1, keepdims=True)
        m_ref[...] = m
        s_ref[...] = jnp.sum(accs_ref[...] * jnp.exp(accm_ref[...] - m),
                             axis=-1, keepdims=True)


def _logprob_kernel(x_ref, m_ref, s_ref, out_ref):
    out_ref[...] = x_ref[...] - m_ref[...] - jnp.log(s_ref[...])


NW = 32
RPW = B // NW
NV = V // 16


def _keys_of(bits):
    return jnp.where(bits < 0, jnp.bitwise_xor(bits, jnp.int32(0x7FFFFFFF)),
                     bits)


def _scal(v):
    return jnp.max(v)


def _scan_desc(hist_ref, nbuckets, target):
    nv = nbuckets // 16
    lane = lax.iota(jnp.int32, 16)

    def body(i, carry):
        found, bket, c_above, c_prev = carry
        b0 = (nv - 1 - i) * 16
        h = hist_ref[pl.ds(b0, 16)]
        rh = lax.rev(h, (0,))
        rc = plsc.cumsum(rh)
        cum = rc + c_prev
        ge = cum >= target
        cnt_vreg = _scal(rc)
        crossed = jnp.logical_and(found == 0, _scal(jnp.where(ge, 1, 0)) > 0)
        jstar = jnp.sum(jnp.where(ge, 0, 1))
        bnew = b0 + 15 - jstar
        hval = jnp.sum(jnp.where(lane == jstar, rh, 0))
        cum_at = jnp.sum(jnp.where(lane == jstar, cum, 0))
        bket = jnp.where(crossed, bnew, bket)
        c_above = jnp.where(crossed, cum_at - hval, c_above)
        found = jnp.where(crossed, 1, found)
        return (found, bket, c_above, c_prev + cnt_vreg)

    _, bket, c_above, _ = lax.fori_loop(0, nv, body,
                                        (jnp.int32(0), jnp.int32(0),
                                         jnp.int32(0), jnp.int32(0)))
    return bket, c_above


U = 5
NG = NV // U


def _sc_select_body(logits_hbm, vals_hbm, idx_hbm,
                    row_v, hist_v, hist3_v, cv_v, ci_v):
    wid = lax.axis_index("s") * 2 + lax.axis_index("c")
    lane = lax.iota(jnp.int32, 16)
    ones = jnp.full((16,), 1, jnp.int32)
    zeros16 = jnp.zeros((16,), jnp.int32)

    def _kv(off):
        v = row_v[pl.ds(off, 16)]
        return v, _keys_of(lax.bitcast_convert_type(v, jnp.int32))

    for r in range(RPW):
        row = wid * RPW + r
        pltpu.sync_copy(logits_hbm.at[row], row_v)

        def clr(i, _):
            for u in range(16):
                hist_v[pl.ds(i * 256 + u * 16, 16)] = zeros16
            return 0
        lax.fori_loop(0, 16, clr, 0)

        for u in range(16):
            hist3_v[pl.ds(u * 16, 16)] = zeros16

        @plsc.parallel_loop(0, NV, unroll=8)
        def _pa(i):
            _, k = _kv(i * 16)
            plsc.addupdate_scatter(hist_v, [(k >> 20) + 2048], ones)

        beta, c_above = _scan_desc(hist_v, 4096, jnp.int32(K))
        need_in = K - c_above

        lax.fori_loop(0, 16, clr, 0)

        @plsc.parallel_loop(0, NV, unroll=8)
        def _pb(i):
            _, k = _kv(i * 16)
            mask = ((k >> 20) + 2048) == beta
            plsc.addupdate_scatter(hist_v, [(k >> 8) & 0xFFF], ones,
                                   mask=mask)

        gamma, c_above2 = _scan_desc(hist_v, 4096, need_in)
        need2 = need_in - c_above2
        kh = ((beta - 2048) << 12) | gamma

        @plsc.parallel_loop(0, NV, unroll=8)
        def _pc(i):
            _, k = _kv(i * 16)
            mask = (k >> 8) == kh
            plsc.addupdate_scatter(hist3_v, [k & 0xFF], ones, mask=mask)

        delta, c_above3 = _scan_desc(hist3_v, 256, need2)
        tau = (kh << 8) | delta
        c_gt = c_above + c_above2 + c_above3

        zeros_sp = jnp.zeros((16,), jnp.int32)

        @plsc.parallel_loop(0, NV, unroll=8,
                            carry=(zeros_sp, zeros_sp + c_gt))
        def _pd(i, carry):
            ogt, oeq = carry
            off = i * 16
            v, k = _kv(off)
            idxv = lane + off
            mgt = k > tau
            meq = k == tau
            rem = K - oeq

            pcg = plsc.cumsum(jnp.where(mgt, 1, 0))
            destg = ogt + pcg - 1
            plsc.store_scatter(cv_v, [destg], v, mask=mgt)
            plsc.store_scatter(ci_v, [destg], idxv, mask=mgt)
            pce = plsc.cumsum(jnp.where(meq, 1, 0))
            allowed = jnp.logical_and(meq, pce <= rem)
            deste = oeq + pce - 1
            plsc.store_scatter(cv_v, [deste], v, mask=allowed)
            plsc.store_scatter(ci_v, [deste], idxv, mask=allowed)

            cnt_gt = plsc.all_reduce_population_count(mgt)
            cnt_eq = plsc.all_reduce_population_count(meq)
            return (ogt + cnt_gt, oeq + jnp.minimum(cnt_eq, rem))

        pltpu.sync_copy(cv_v, vals_hbm.at[row])
        pltpu.sync_copy(ci_v, idx_hbm.at[row])


def _sc_select(logits):
    mesh = plsc.VectorSubcoreMesh(core_axis_name="c", subcore_axis_name="s",
                                  num_cores=2, num_subcores=16)
    return pl.kernel(
        _sc_select_body,
        out_type=(jax.ShapeDtypeStruct((B, K), jnp.float32),
                  jax.ShapeDtypeStruct((B, K), jnp.int32)),
        mesh=mesh,
        compiler_params=pltpu.CompilerParams(use_tc_tiling_on_sc=False,
                                             needs_layout_passes=False),
        scratch_types=[
            pltpu.VMEM((V,), jnp.float32),
            pltpu.VMEM((4096,), jnp.int32),
            pltpu.VMEM((256,), jnp.int32),
            pltpu.VMEM((K,), jnp.float32),
            pltpu.VMEM((K,), jnp.int32),
        ],
    )(logits)


def _sample_kernel(vals_ref, idx_ref, m_ref, s_ref, tk_ref, tp_ref, mp_ref,
                   u_ref, tok_ref):
    x = vals_ref[...]
    idx = idx_ref[...]
    p = jnp.exp(x - m_ref[...]) / s_ref[...]
    iota = lax.broadcasted_iota(jnp.int32, (B, K), 1)

    k = 2
    while k <= K:
        j = k // 2
        while j >= 1:
            pt_p_lo = pltpu.roll(p, K - j, 1)
            pt_p_hi = pltpu.roll(p, j, 1)
            pt_i_lo = pltpu.roll(idx, K - j, 1)
            pt_i_hi = pltpu.roll(idx, j, 1)
            is_lo = (iota & j) == 0
            pt_p = jnp.where(is_lo, pt_p_lo, pt_p_hi)
            pt_i = jnp.where(is_lo, pt_i_lo, pt_i_hi)
            asc = (iota & k) == 0
            before = (p > pt_p) | ((p == pt_p) & (idx < pt_i))
            sel = (asc == is_lo) == before
            p = jnp.where(sel, p, pt_p)
            idx = jnp.where(sel, idx, pt_i)
            j //= 2
        k *= 2

    c = p
    t = 1
    while t < K:
        c = c + jnp.where(iota >= t, pltpu.roll(c, t, 1), 0.0)
        t *= 2

    p1 = jnp.where(iota >= tk_ref[...], 0.0, p)
    p2 = jnp.where(c - p1 > tp_ref[...], 0.0, p1)
    th = p2[:, 0:1] * mp_ref[...]
    p3 = jnp.where(p2 < th, 0.0, p2)

    g = -jnp.log(-jnp.log(u_ref[...]))
    scores = jnp.log(p3 + 1e-30) + g
    amax = jnp.argmax(scores, axis=-1)
    tok = jnp.sum(jnp.where(iota == amax[:, None], idx, 0), axis=-1,
                  dtype=jnp.int32)
    tok_ref[...] = tok[:, None]


def _run_sample(vals, idx, m, s, top_ks, top_ps, min_ps):
    u = jnp.asarray(_U_SLAB)
    full = lambda shape: pl.BlockSpec(shape, lambda: (0, 0))
    return pl.pallas_call(
        _sample_kernel,
        in_specs=[full((B, K)), full((B, K)), full((B, 1)), full((B, 1)),
                  full((B, 1)), full((B, 1)), full((B, 1)), full((B, K))],
        out_specs=full((B, 1)),
        out_shape=jax.ShapeDtypeStruct((B, 1), jnp.int32),
    )(vals, idx, m, s, top_ks[:, None].astype(jnp.int32),
      top_ps[:, None], min_ps[:, None], u)


def kernel(logits, top_ks, top_ps, min_ps):
    m, s = pl.pallas_call(
        _stats_kernel,
        grid=(NCHUNK,),
        in_specs=[pl.BlockSpec((B, CHUNK), lambda i: (0, i))],
        out_specs=[pl.BlockSpec((B, 1), lambda i: (0, 0)),
                   pl.BlockSpec((B, 1), lambda i: (0, 0))],
        out_shape=[jax.ShapeDtypeStruct((B, 1), jnp.float32),
                   jax.ShapeDtypeStruct((B, 1), jnp.float32)],
        scratch_shapes=[pltpu.VMEM((B, 128), jnp.float32),
                        pltpu.VMEM((B, 128), jnp.float32)],
    )(logits)

    logprobs = pl.pallas_call(
        _logprob_kernel,
        grid=(NCHUNK,),
        in_specs=[pl.BlockSpec((B, CHUNK), lambda i: (0, i)),
                  pl.BlockSpec((B, 1), lambda i: (0, 0)),
                  pl.BlockSpec((B, 1), lambda i: (0, 0))],
        out_specs=pl.BlockSpec((B, CHUNK), lambda i: (0, i)),
        out_shape=jax.ShapeDtypeStruct((B, V), jnp.float32),
    )(logits, m, s)

    vals, idx = _sc_select(logits)
    tokens = _run_sample(vals, idx, m, s, top_ks, top_ps, min_ps)
    return (tokens.reshape(-1), logprobs)
